# Initial kernel scaffold; baseline (speedup 1.0000x reference)
import jax
import jax.numpy as jnp
from jax.experimental import pallas as pl


def kernel(x, edge_index, batch, params):
    raise NotImplementedError("write your pallas kernel here")


# speedup vs baseline: 8.1069x; 8.1069x over previous
import functools

import jax
import jax.numpy as jnp
from jax import lax
from jax.experimental import pallas as pl
from jax.experimental.pallas import tpu as pltpu
from jax.experimental.pallas import tpu_sc as plsc

_N = 10000
_E = 160000
_NG = 64
_LAYER_DIMS = [(128, 4, 128), (512, 1, 128), (128, 1, 128)]

_NROWS = 10240
_WACC = 144
_NWORK = 32
_EPAD = 170496
_CHUNK = _EPAD // _NWORK
_BLK = 16
_NBLK = _CHUNK // _BLK
_RPT = _NROWS // 16

_GDN = lax.GatherDimensionNumbers(offset_dims=(), collapsed_slice_dims=(0,),
                                  start_index_map=(0,))


def _shuffle16(v, idx):
    return lax.gather(v, idx[:, None], _GDN, slice_sizes=(1,),
                      mode=lax.GatherScatterMode.PROMISE_IN_BOUNDS)


_SROWS = 80


def _edge_pass(xlf, xrf, src, dst, att_h, heads, hd):
    mesh = plsc.VectorSubcoreMesh(core_axis_name="c", subcore_axis_name="s")

    def body(xl_hbm, xr_hbm, src_hbm, dst_hbm, att_hbm, zer_hbm, out_hbm,
             sout_hbm, srcv, dstv, attv, xlb, xrb, wb, wb2, acc, sacc,
             sem1, sem2):
        cid = lax.axis_index("c")
        tid = lax.axis_index("s")
        wid = cid * 16 + tid
        pltpu.sync_copy(zer_hbm, acc.at[pl.ds(tid * _RPT, _RPT)])

        @pl.when(tid == 0)
        def _():
            pltpu.sync_copy(zer_hbm.at[pl.ds(0, _SROWS)], sacc)

        pltpu.sync_copy(src_hbm.at[pl.ds(wid * _CHUNK, _CHUNK)], srcv)
        pltpu.sync_copy(dst_hbm.at[pl.ds(wid * _CHUNK, _CHUNK)], dstv)
        pltpu.sync_copy(att_hbm, attv)
        plsc.subcore_barrier()

        lane = lax.iota(jnp.int32, 16)
        zv = jnp.zeros((16,), jnp.float32)

        def blk(b, carry):
            base = b * _BLK
            sv = srcv[pl.ds(base, _BLK)]
            dv = dstv[pl.ds(base, _BLK)]
            dg = jnp.minimum(dv, _N - 1)
            if heads > 1:
                slr = sv * heads + hd
                dlr = dg * heads + hd
            else:
                slr = sv
                dlr = dg
            cp1 = pltpu.async_copy(xl_hbm.at[slr], xlb, sem1)
            cp2 = pltpu.async_copy(xr_hbm.at[dlr], xrb, sem2)
            cp1.wait()
            cp2.wait()
            for e in range(_BLK):
                accv = jnp.zeros((16,), jnp.float32)
                avals = []
                for d in range(8):
                    a = xlb[e, pl.ds(d * 16, 16)]
                    r = xrb[e, pl.ds(d * 16, 16)]
                    t = a + r
                    t = jnp.maximum(t, 0.2 * t)
                    accv = accv + t * attv[pl.ds(d * 16, 16)]
                    avals.append(a)
                for k in (1, 2, 4, 8):
                    accv = accv + _shuffle16(accv, lane ^ k)
                pe = jnp.exp(accv)
                for d in range(8):
                    wb[e, pl.ds(d * 16, 16)] = avals[d] * pe
                dvi = dv[e]
                dmod = lax.bitwise_and(dvi, 127)
                voff = lax.bitwise_and(dmod, 0x70)
                lt = lax.bitwise_and(dmod, 15)
                for d in range(8):
                    wb2[e, pl.ds(d * 16, 16)] = zv
                wb2[e, pl.ds(voff, 16)] = jnp.where(lane == lt, pe, 0.0)
            c1 = pltpu.async_copy(wb, acc.at[dv], sem1, add=True)
            c2 = pltpu.async_copy(wb2, sacc.at[lax.shift_right_logical(dv, 7)],
                                  sem2, add=True)
            c1.wait()
            c2.wait()
            return carry

        lax.fori_loop(0, _NBLK, blk, 0)
        plsc.subcore_barrier()
        pltpu.sync_copy(acc.at[pl.ds(tid * _RPT, _RPT)],
                        out_hbm.at[cid, pl.ds(tid * _RPT, _RPT)])

        @pl.when(tid == 0)
        def _():
            pltpu.sync_copy(sacc, sout_hbm.at[cid])

    fn = pl.kernel(
        body,
        out_type=(jax.ShapeDtypeStruct((2, _NROWS, 128), jnp.float32),
                  jax.ShapeDtypeStruct((2, _SROWS, 128), jnp.float32)),
        mesh=mesh,
        scratch_types=[
            pltpu.VMEM((_CHUNK,), jnp.int32),
            pltpu.VMEM((_CHUNK,), jnp.int32),
            pltpu.VMEM((128,), jnp.float32),
            pltpu.VMEM((_BLK, 128), jnp.float32),
            pltpu.VMEM((_BLK, 128), jnp.float32),
            pltpu.VMEM((_BLK, 128), jnp.float32),
            pltpu.VMEM((_BLK, 128), jnp.float32),
            pltpu.VMEM_SHARED((_NROWS, 128), jnp.float32),
            pltpu.VMEM_SHARED((_SROWS, 128), jnp.float32),
            pltpu.SemaphoreType.DMA,
            pltpu.SemaphoreType.DMA,
        ],
    )
    zer = jnp.zeros((_RPT, 128), jnp.float32)
    return fn(xlf, xrf, src, dst, att_h, zer)


def _lin2(h, wl, bl, wr, br):
    n, di = h.shape
    do = wl.shape[1]
    r = 400
    g = n // r

    def body(h_ref, wl_ref, bl_ref, wr_ref, br_ref, ol_ref, or_ref):
        hb = h_ref[...]
        ol_ref[...] = jnp.dot(hb, wl_ref[...],
                              preferred_element_type=jnp.float32) + bl_ref[...]
        or_ref[...] = jnp.dot(hb, wr_ref[...],
                              preferred_element_type=jnp.float32) + br_ref[...]

    return pl.pallas_call(
        body,
        grid=(g,),
        in_specs=[
            pl.BlockSpec((r, di), lambda i: (i, 0)),
            pl.BlockSpec((di, do), lambda i: (0, 0)),
            pl.BlockSpec((1, do), lambda i: (0, 0)),
            pl.BlockSpec((di, do), lambda i: (0, 0)),
            pl.BlockSpec((1, do), lambda i: (0, 0)),
        ],
        out_specs=[
            pl.BlockSpec((r, do), lambda i: (i, 0)),
            pl.BlockSpec((r, do), lambda i: (i, 0)),
        ],
        out_shape=[jax.ShapeDtypeStruct((n, do), jnp.float32)] * 2,
    )(h, wl, bl.reshape(1, -1), wr, br.reshape(1, -1))


def _post(accs, svecs, bias, gamma, beta, res):
    nh = len(accs)
    do = nh * 128
    r = 400
    g = _N // r

    def body(*refs):
        acc_refs = refs[:nh]
        s_refs = refs[nh:2 * nh]
        b_ref, g_ref, be_ref = refs[2 * nh:2 * nh + 3]
        if res is not None:
            r_ref = refs[2 * nh + 3]
            o_ref = refs[2 * nh + 4]
        else:
            o_ref = refs[2 * nh + 3]
        outs = []
        for a_ref, s_ref in zip(acc_refs, s_refs):
            a = a_ref[0] + a_ref[1]
            outs.append(a / (s_ref[...] + 1e-16))
        h = jnp.concatenate(outs, axis=1) if nh > 1 else outs[0]
        h = h + b_ref[...]
        mu = jnp.mean(h, axis=-1, keepdims=True)
        var = jnp.mean((h - mu) ** 2, axis=-1, keepdims=True)
        y = (h - mu) / jnp.sqrt(var + 1e-5) * g_ref[...] + be_ref[...]
        y = jnp.maximum(y, 0.0)
        if res is not None:
            y = y + r_ref[...]
        o_ref[...] = y

    in_specs = [pl.BlockSpec((2, r, 128), lambda i: (0, i, 0))
                for _ in range(nh)]
    in_specs += [pl.BlockSpec((r, 1), lambda i: (i, 0)) for _ in range(nh)]
    in_specs += [pl.BlockSpec((1, do), lambda i: (0, 0))] * 3
    args = list(accs) + list(svecs) + [bias.reshape(1, -1),
                                       gamma.reshape(1, -1),
                                       beta.reshape(1, -1)]
    if res is not None:
        in_specs.append(pl.BlockSpec((r, do), lambda i: (i, 0)))
        args.append(res)

    return pl.pallas_call(
        body,
        grid=(g,),
        in_specs=in_specs,
        out_specs=pl.BlockSpec((r, do), lambda i: (i, 0)),
        out_shape=jax.ShapeDtypeStruct((_N, do), jnp.float32),
    )(*args)


def _pool_mlp(h, batg, wh1, bh1, wh2, bh2):
    r = 400
    g = _N // r

    def body(h_ref, b_ref, w1_ref, b1_ref, w2_ref, b2_ref, o_ref, z_scr, c_scr):
        i = pl.program_id(0)

        @pl.when(i == 0)
        def _():
            z_scr[...] = jnp.zeros_like(z_scr)
            c_scr[...] = jnp.zeros_like(c_scr)

        bb = b_ref[0, 0, :]
        gid = lax.broadcasted_iota(jnp.int32, (_NG, r), 0)
        onehot = (gid == bb[None, :]).astype(jnp.float32)
        z_scr[...] += jnp.dot(onehot, h_ref[...],
                              preferred_element_type=jnp.float32)
        c_scr[...] += jnp.sum(onehot, axis=1, keepdims=True)

        @pl.when(i == g - 1)
        def _():
            zm = z_scr[...] / jnp.maximum(c_scr[...], 1.0)
            a = jnp.maximum(
                jnp.dot(zm, w1_ref[...],
                        preferred_element_type=jnp.float32) + b1_ref[...], 0.0)
            o_ref[...] = jnp.dot(a, w2_ref[...],
                                 preferred_element_type=jnp.float32) + b2_ref[...]

    return pl.pallas_call(
        body,
        grid=(g,),
        in_specs=[
            pl.BlockSpec((r, 128), lambda i: (i, 0)),
            pl.BlockSpec((1, 1, r), lambda i: (i, 0, 0)),
            pl.BlockSpec((128, 128), lambda i: (0, 0)),
            pl.BlockSpec((1, 128), lambda i: (0, 0)),
            pl.BlockSpec((128, 1), lambda i: (0, 0)),
            pl.BlockSpec((1, 1), lambda i: (0, 0)),
        ],
        out_specs=pl.BlockSpec((_NG, 1), lambda i: (0, 0)),
        out_shape=jax.ShapeDtypeStruct((_NG, 1), jnp.float32),
        scratch_shapes=[
            pltpu.VMEM((_NG, 128), jnp.float32),
            pltpu.VMEM((_NG, 128), jnp.float32),
        ],
    )(h, batg, wh1, bh1.reshape(1, -1), wh2, bh2.reshape(1, -1))


def kernel(x, edge_index, batch, params):
    loop = jnp.arange(_N, dtype=jnp.int32)
    npad = _EPAD - _E - _N
    src = jnp.concatenate([edge_index[0], loop,
                           jnp.zeros((npad,), jnp.int32)])
    dst = jnp.concatenate([edge_index[1], loop,
                           jnp.full((npad,), _N, jnp.int32)])
    batg = batch.reshape(_N // 400, 1, 400)

    h = x
    for l, (i_d, heads, o_d) in enumerate(_LAYER_DIMS):
        xl, xr = _lin2(h, params[f'Wl{l}'], params[f'bl{l}'],
                       params[f'Wr{l}'], params[f'br{l}'])
        xlf = xl.reshape(_N * heads, o_d)
        xrf = xr.reshape(_N * heads, o_d)
        accs, svecs = [], []
        for hd in range(heads):
            acc, sacc = _edge_pass(xlf, xrf, src, dst,
                                   params[f'att{l}'][hd], heads, hd)
            accs.append(acc)
            svecs.append((sacc[0] + sacc[1]).reshape(_SROWS * 128)[:_N]
                         .reshape(_N, 1))
        res = h if h.shape[1] == heads * o_d else None
        h = _post(accs, svecs, params[f'bias{l}'], params[f'gamma{l}'],
                  params[f'beta{l}'], res)
    return _pool_mlp(h, batg, params['Wh1'], params['bh1'],
                     params['Wh2'], params['bh2'])
